# Initial kernel scaffold; baseline (speedup 1.0000x reference)
import functools

import jax
import jax.numpy as jnp
from jax import lax
from jax.experimental import pallas as pl
from jax.experimental.pallas import tpu as pltpu
from jax.experimental.pallas import tpu_sc as plsc


def _matmul_halves(h, W_T):
    N, K = h.shape
    DO = W_T.shape[1]
    DH = DO // 2
    RB = 1000

    def mm_body(h_ref, w_ref, o_ref):
        o_ref[0] = jnp.dot(h_ref[...], w_ref[...],
                           preferred_element_type=jnp.float32)

    return pl.pallas_call(
        mm_body,
        grid=(2, N // RB),
        in_specs=[
            pl.BlockSpec((RB, K), lambda c, j: (j, 0)),
            pl.BlockSpec((K, DH), lambda c, j: (0, c)),
        ],
        out_specs=pl.BlockSpec((1, RB, DH), lambda c, j: (c, j, 0)),
        out_shape=jax.ShapeDtypeStruct((2, N, DH), jnp.float32),
    )(h, W_T)


def _edge_aggregate(h2p, row, col, attn, N, DO):
    E = row.shape[0]
    DH = DO // 2
    NT = 16
    ROWS_PT = N // NT
    EDGES_PT = E // NT
    K = 80
    NCHUNK = EDGES_PT // K
    ZR = 125
    NQ = DH // 16

    mesh = plsc.VectorSubcoreMesh(core_axis_name="c", subcore_axis_name="s")

    @functools.partial(
        pl.kernel,
        mesh=mesh,
        out_type=jax.ShapeDtypeStruct((N, DO), jnp.float32),
        scratch_types=[
            pltpu.VMEM((K,), jnp.int32),
            pltpu.VMEM((K,), jnp.int32),
            pltpu.VMEM((K,), jnp.float32),
            pltpu.VMEM((K, DH), jnp.float32),
            pltpu.VMEM((ZR, DH), jnp.float32),
            pltpu.VMEM_SHARED((N, DH), jnp.float32),
            pltpu.VMEM_SHARED((N, DH), jnp.float32),
            pltpu.SemaphoreType.DMA,
        ],
    )
    def agg(h2_hbm, row_hbm, col_hbm, attn_hbm, out_hbm,
            col_v, row_v, attn_v, msg_v, zero_v, h2_s, acc_s, sem):
        c = lax.axis_index("c")
        s = lax.axis_index("s")
        r_lo = s * ROWS_PT

        pltpu.sync_copy(h2_hbm.at[c, pl.ds(r_lo, ROWS_PT)],
                        h2_s.at[pl.ds(r_lo, ROWS_PT)])

        zvec = jnp.zeros((16,), jnp.float32)

        def zero_body(i, carry):
            for q in range(NQ):
                zero_v[i, pl.ds(q * 16, 16)] = zvec
            return carry

        lax.fori_loop(0, ZR, zero_body, 0)
        for b in range(ROWS_PT // ZR):
            pltpu.sync_copy(zero_v, acc_s.at[pl.ds(r_lo + b * ZR, ZR)])

        plsc.subcore_barrier()

        base = s * EDGES_PT

        def chunk_body(j, carry):
            off = base + j * K
            pltpu.sync_copy(col_hbm.at[pl.ds(off, K)], col_v)
            pltpu.sync_copy(row_hbm.at[pl.ds(off, K)], row_v)
            pltpu.sync_copy(attn_hbm.at[pl.ds(off, K)], attn_v)
            pltpu.async_copy(h2_s.at[col_v], msg_v, sem).wait()
            for e in range(K):
                sp = plsc.load_gather(
                    attn_v, [jnp.full((16,), e, jnp.int32)])
                for q in range(NQ):
                    sl = pl.ds(q * 16, 16)
                    msg_v[e, sl] = msg_v[e, sl] * sp
            pltpu.sync_copy(msg_v, acc_s.at[row_v], add=True)
            return carry

        lax.fori_loop(0, NCHUNK, chunk_body, 0)

        plsc.subcore_barrier()

        for b in range(ROWS_PT // ZR):
            r0 = r_lo + b * ZR
            pltpu.sync_copy(acc_s.at[pl.ds(r0, ZR)],
                            out_hbm.at[pl.ds(r0, ZR), pl.ds(c * DH, DH)])

    return agg(h2p, row, col, attn)


def kernel(h, edge_index, attn, W_T):
    N = h.shape[0]
    DO = W_T.shape[1]
    row = edge_index[0].astype(jnp.int32)
    col = edge_index[1].astype(jnp.int32)
    attn = attn.astype(jnp.float32)
    h2p = _matmul_halves(h.astype(jnp.float32), W_T.astype(jnp.float32))
    return _edge_aggregate(h2p, row, col, attn, N, DO)


# speedup vs baseline: 3.6286x; 3.6286x over previous
import functools

import jax
import jax.numpy as jnp
from jax import lax
from jax.experimental import pallas as pl
from jax.experimental.pallas import tpu as pltpu
from jax.experimental.pallas import tpu_sc as plsc


def _matmul(h, W_T):
    N, K = h.shape
    DO = W_T.shape[1]
    RB = 1000

    def mm_body(h_ref, w_ref, o_ref):
        o_ref[...] = jnp.dot(h_ref[...], w_ref[...],
                             preferred_element_type=jnp.float32)

    return pl.pallas_call(
        mm_body,
        grid=(N // RB,),
        in_specs=[
            pl.BlockSpec((RB, K), lambda j: (j, 0)),
            pl.BlockSpec((K, DO), lambda j: (0, 0)),
        ],
        out_specs=pl.BlockSpec((RB, DO), lambda j: (j, 0)),
        out_shape=jax.ShapeDtypeStruct((N, DO), jnp.float32),
    )(h, W_T)


def _edge_aggregate(h2, row, col, attn, N_PAD, DO):
    E = row.shape[0]
    NT = 16
    R_COUNT = N_PAD // NT
    EDGES_PT = E // (2 * NT)
    K = 80
    NCHUNK = EDGES_PT // K
    ZR = 128
    NQ = DO // 16

    mesh = plsc.VectorSubcoreMesh(core_axis_name="c", subcore_axis_name="s")

    @functools.partial(
        pl.kernel,
        mesh=mesh,
        out_type=jax.ShapeDtypeStruct((2, N_PAD, DO), jnp.float32),
        scratch_types=[
            pltpu.VMEM((K,), jnp.int32),
            pltpu.VMEM((K,), jnp.int32),
            pltpu.VMEM((K,), jnp.float32),
            pltpu.VMEM((K, DO), jnp.float32),
            pltpu.VMEM((ZR, DO), jnp.float32),
            pltpu.VMEM_SHARED((N_PAD, DO), jnp.float32),
            pltpu.SemaphoreType.DMA,
        ],
        compiler_params=pltpu.CompilerParams(needs_layout_passes=False),
    )
    def agg(h2_hbm, row_hbm, col_hbm, attn_hbm, out_hbm,
            col_v, row_v, attn_v, msg_v, zero_v, acc_s, sem):
        c = lax.axis_index("c")
        s = lax.axis_index("s")
        r_lo = s * R_COUNT

        zvec = jnp.zeros((16,), jnp.float32)

        def zero_body(i, carry):
            for q in range(NQ):
                zero_v[i, pl.ds(q * 16, 16)] = zvec
            return carry

        lax.fori_loop(0, ZR, zero_body, 0)
        for b in range(R_COUNT // ZR):
            pltpu.sync_copy(zero_v, acc_s.at[pl.ds(r_lo + b * ZR, ZR)])

        plsc.subcore_barrier()

        base = (c * NT + s) * EDGES_PT

        def chunk_body(j, carry):
            off = base + j * K
            pltpu.sync_copy(col_hbm.at[pl.ds(off, K)], col_v)
            pltpu.sync_copy(row_hbm.at[pl.ds(off, K)], row_v)
            pltpu.sync_copy(attn_hbm.at[pl.ds(off, K)], attn_v)
            pltpu.async_copy(h2_hbm.at[col_v], msg_v, sem).wait()

            def scale_body(e, carry):
                sp = plsc.load_gather(
                    attn_v, [jnp.full((16,), 0, jnp.int32) + e])
                for q in range(NQ):
                    sl = pl.ds(q * 16, 16)
                    msg_v[e, sl] = msg_v[e, sl] * sp
                return carry

            lax.fori_loop(0, K, scale_body, 0)
            pltpu.sync_copy(msg_v, acc_s.at[row_v], add=True)
            return carry

        lax.fori_loop(0, NCHUNK, chunk_body, 0)

        plsc.subcore_barrier()

        for b in range(R_COUNT // ZR):
            r0 = r_lo + b * ZR
            pltpu.sync_copy(acc_s.at[pl.ds(r0, ZR)],
                            out_hbm.at[c, pl.ds(r0, ZR)])

    return agg(h2, row, col, attn)


def _combine(out_p):
    _, N_PAD, DO = out_p.shape
    RB = 640

    def add_body(i_ref, o_ref):
        o_ref[...] = i_ref[0] + i_ref[1]

    return pl.pallas_call(
        add_body,
        grid=(N_PAD // RB,),
        in_specs=[pl.BlockSpec((2, RB, DO), lambda j: (0, j, 0))],
        out_specs=pl.BlockSpec((RB, DO), lambda j: (j, 0)),
        out_shape=jax.ShapeDtypeStruct((N_PAD, DO), jnp.float32),
    )(out_p)


def kernel(h, edge_index, attn, W_T):
    N = h.shape[0]
    DO = W_T.shape[1]
    N_PAD = 10240
    row = edge_index[0].astype(jnp.int32)
    col = edge_index[1].astype(jnp.int32)
    attn = attn.astype(jnp.float32)
    h2 = _matmul(h.astype(jnp.float32), W_T.astype(jnp.float32))
    out_p = _edge_aggregate(h2, row, col, attn, N_PAD, DO)
    return _combine(out_p)[:N]


# speedup vs baseline: 7.3405x; 2.0230x over previous
import functools

import jax
import jax.numpy as jnp
from jax import lax
from jax.experimental import pallas as pl
from jax.experimental.pallas import tpu as pltpu
from jax.experimental.pallas import tpu_sc as plsc


def _matmul(h, W_T):
    N, K = h.shape
    DO = W_T.shape[1]
    RB = 1000

    def mm_body(h_ref, w_ref, o_ref):
        o_ref[...] = jnp.dot(h_ref[...], w_ref[...],
                             preferred_element_type=jnp.float32)

    return pl.pallas_call(
        mm_body,
        grid=(N // RB,),
        in_specs=[
            pl.BlockSpec((RB, K), lambda j: (j, 0)),
            pl.BlockSpec((K, DO), lambda j: (0, 0)),
        ],
        out_specs=pl.BlockSpec((RB, DO), lambda j: (j, 0)),
        out_shape=jax.ShapeDtypeStruct((N, DO), jnp.float32),
    )(h, W_T)


def _edge_aggregate(h2, row3, col3, attn3, N_PAD, DO):
    NW, NSC, SCC, K = row3.shape
    NT = 16
    R_COUNT = N_PAD // NT
    ZR = 128
    NQ = DO // 16
    NPAIR = (SCC - 3) // 2

    mesh = plsc.VectorSubcoreMesh(core_axis_name="c", subcore_axis_name="s")

    @functools.partial(
        pl.kernel,
        mesh=mesh,
        out_type=jax.ShapeDtypeStruct((2, N_PAD, DO), jnp.float32),
        scratch_types=[
            pltpu.VMEM((SCC, K), jnp.int32),
            pltpu.VMEM((SCC, K), jnp.int32),
            pltpu.VMEM((SCC, K), jnp.float32),
            pltpu.VMEM((2, K, DO), jnp.float32),
            pltpu.VMEM_SHARED((N_PAD, DO), jnp.float32),
            pltpu.SemaphoreType.DMA,
            pltpu.SemaphoreType.DMA,
            pltpu.SemaphoreType.DMA,
            pltpu.SemaphoreType.DMA,
        ],
        compiler_params=pltpu.CompilerParams(needs_layout_passes=False),
    )
    def agg(h2_hbm, row_hbm, col_hbm, attn_hbm, out_hbm,
            col_b, row_b, attn_b, msg_v, acc_s, g0, g1, s0, s1):
        c = lax.axis_index("c")
        s = lax.axis_index("s")
        w = c * NT + s
        r_lo = s * R_COUNT
        gsem = (g0, g1)
        ssem = (s0, s1)

        zvec = jnp.zeros((16,), jnp.float32)

        def zero_body(i, carry):
            for q in range(NQ):
                msg_v[0, i, pl.ds(q * 16, 16)] = zvec
            return carry

        lax.fori_loop(0, K, zero_body, 0)
        for b in range(R_COUNT // K):
            pltpu.sync_copy(msg_v.at[0], acc_s.at[pl.ds(r_lo + b * K, K)])

        plsc.subcore_barrier()

        def start_gather(ci, b):
            pltpu.async_copy(h2_hbm.at[col_b.at[ci]], msg_v.at[b], gsem[b])

        def wait_gather(ci, b):
            pltpu.make_async_copy(
                h2_hbm.at[col_b.at[ci]], msg_v.at[b], gsem[b]).wait()

        def start_scatter(ci, b):
            pltpu.async_copy(msg_v.at[b], acc_s.at[row_b.at[ci]], ssem[b],
                             add=True)

        def wait_scatter(ci, b):
            pltpu.make_async_copy(
                msg_v.at[b], acc_s.at[row_b.at[ci]], ssem[b]).wait()

        def scale(ci, b):
            def group(g, carry):
                e0 = g * 16
                for l in range(16):
                    e = e0 + l
                    sp = plsc.load_gather(
                        attn_b,
                        [jnp.full((16,), 0, jnp.int32) + ci,
                         jnp.full((16,), 0, jnp.int32) + e])
                    for q in range(NQ):
                        sl = pl.ds(q * 16, 16)
                        msg_v[b, e, sl] = msg_v[b, e, sl] * sp
                return carry

            lax.fori_loop(0, K // 16, group, 0)

        def superchunk_body(scj, carry):
            pltpu.sync_copy(col_hbm.at[w, scj], col_b)
            pltpu.sync_copy(row_hbm.at[w, scj], row_b)
            pltpu.sync_copy(attn_hbm.at[w, scj], attn_b)

            start_gather(0, 0)
            start_gather(1, 1)

            def pair_body(j2, carry):
                c0 = 2 * j2
                wait_gather(c0, 0)
                scale(c0, 0)
                start_scatter(c0, 0)
                wait_gather(c0 + 1, 1)
                scale(c0 + 1, 1)
                start_scatter(c0 + 1, 1)
                wait_scatter(c0, 0)
                start_gather(c0 + 2, 0)
                wait_scatter(c0 + 1, 1)
                start_gather(c0 + 3, 1)
                return carry

            lax.fori_loop(0, NPAIR, pair_body, 0)

            t0 = SCC - 3
            wait_gather(t0, 0)
            scale(t0, 0)
            start_scatter(t0, 0)
            wait_gather(t0 + 1, 1)
            scale(t0 + 1, 1)
            start_scatter(t0 + 1, 1)
            wait_scatter(t0, 0)
            start_gather(t0 + 2, 0)
            wait_gather(t0 + 2, 0)
            scale(t0 + 2, 0)
            start_scatter(t0 + 2, 0)
            wait_scatter(t0 + 1, 1)
            wait_scatter(t0 + 2, 0)
            return carry

        lax.fori_loop(0, NSC, superchunk_body, 0)

        plsc.subcore_barrier()

        for b in range(R_COUNT // ZR):
            r0 = r_lo + b * ZR
            pltpu.sync_copy(acc_s.at[pl.ds(r0, ZR)],
                            out_hbm.at[c, pl.ds(r0, ZR)])

    return agg(h2, row3, col3, attn3)


def _combine(out_p):
    _, N_PAD, DO = out_p.shape
    RB = 640

    def add_body(i_ref, o_ref):
        o_ref[...] = i_ref[0] + i_ref[1]

    return pl.pallas_call(
        add_body,
        grid=(N_PAD // RB,),
        in_specs=[pl.BlockSpec((2, RB, DO), lambda j: (0, j, 0))],
        out_specs=pl.BlockSpec((RB, DO), lambda j: (j, 0)),
        out_shape=jax.ShapeDtypeStruct((N_PAD, DO), jnp.float32),
    )(out_p)


def kernel(h, edge_index, attn, W_T):
    N = h.shape[0]
    DO = W_T.shape[1]
    E = attn.shape[0]
    N_PAD = 10240
    NW, K, SCC = 32, 80, 25
    NSC = E // (NW * SCC * K)
    row3 = edge_index[0].astype(jnp.int32).reshape(NW, NSC, SCC, K)
    col3 = edge_index[1].astype(jnp.int32).reshape(NW, NSC, SCC, K)
    attn3 = attn.astype(jnp.float32).reshape(NW, NSC, SCC, K)
    h2 = _matmul(h.astype(jnp.float32), W_T.astype(jnp.float32))
    out_p = _edge_aggregate(h2, row3, col3, attn3, N_PAD, DO)
    return _combine(out_p)[:N]
